# baseline (speedup 1.0000x reference)
import functools
import math

import jax
import jax.numpy as jnp
from jax import lax
from jax.experimental import pallas as pl
from jax.experimental.pallas import tpu as pltpu
from jax.experimental.pallas import tpu_sc as plsc

D_MODEL = 64
SCALE = math.sqrt(D_MODEL)

NUM_CORES = 2
NUM_SUBCORES = 16
NW = NUM_CORES * NUM_SUBCORES

BI = 128


def _gather_body(n_seq, xT_hbm, c_hbm, out_hbm, xtb, wb0, wb1, g0, g1,
                 buf0, buf1, sg0, sg1, so0, so1):
  t = lax.axis_index("s") * NUM_CORES + lax.axis_index("c")
  pltpu.sync_copy(xT_hbm.at[:, pl.ds(t * BI, BI)], xtb)

  iota = lax.iota(jnp.int32, 16)
  rvecs = [i0 * 16 + iota for i0 in range(8)]

  def prep(j, wb):
    for l in range(8):
      s = pl.ds(l * 16, 16)
      wb[s] = xtb[j, s] >> 1

  def fire_gather(g, sem):
    pltpu.async_copy(c_hbm.at[g[0]], g[1], sem)

  def drain_gather(g, sem):
    pltpu.make_async_copy(c_hbm.at[g[0]], g[1], sem).wait()

  def assemble(j, g, buf):
    rows = g[1]
    for i0 in range(8):
      rv = rvecs[i0]
      pv = (xtb[j, pl.ds(i0 * 16, 16)] & 1) * D_MODEL
      si = pl.ds(i0 * 16, 16)

      @plsc.parallel_loop(0, D_MODEL, unroll=4)
      def _(k):
        buf[k, si] = plsc.load_gather(rows, [rv, pv + k]) * SCALE

  def fire_out(j, buf, sem):
    for k8 in range(8):
      pltpu.async_copy(buf.at[pl.ds(k8 * 8, 8)],
                       out_hbm.at[j, pl.ds(k8 * 8, 8), pl.ds(t * BI, BI)],
                       sem)

  def drain_out(j, buf, sem):
    for k8 in range(8):
      pltpu.make_async_copy(buf.at[pl.ds(k8 * 8, 8)],
                            out_hbm.at[j, pl.ds(k8 * 8, 8),
                                       pl.ds(t * BI, BI)], sem).wait()

  ga = (wb0, g0)
  gb = (wb1, g1)
  prep(0, wb0)
  fire_gather(ga, sg0)

  def pair(jj, carry):
    j0 = 2 * jj
    j1 = j0 + 1
    prep(j1, wb1)

    @pl.when(jj > 0)
    def _():
      drain_out(j1 - 2, buf1, so1)

    fire_gather(gb, sg1)
    drain_gather(ga, sg0)
    assemble(j0, ga, buf0)
    fire_out(j0, buf0, so0)
    drain_gather(gb, sg1)

    @pl.when(j1 + 1 < n_seq)
    def _():
      prep(j1 + 1, wb0)
      fire_gather(ga, sg0)

    drain_out(j0, buf0, so0)
    assemble(j1, gb, buf1)
    fire_out(j1, buf1, so1)
    return carry

  lax.fori_loop(0, n_seq // 2, pair, 0)
  drain_out(n_seq - 1, buf1, so1)


def kernel(x, table):
  b0, b1 = x.shape
  v = table.shape[0]
  assert b0 == NW * BI and b1 % 2 == 0 and v % 2 == 0
  xT = x.T.astype(jnp.int32)
  table_c = table.reshape(v // 2, 2 * D_MODEL)

  mesh = plsc.VectorSubcoreMesh(core_axis_name="c", subcore_axis_name="s")
  gather = functools.partial(
      pl.kernel,
      mesh=mesh,
      out_type=jax.ShapeDtypeStruct((b1, D_MODEL, b0), jnp.float32),
      scratch_types=[
          pltpu.VMEM((b1, BI), jnp.int32),
          pltpu.VMEM((BI,), jnp.int32),
          pltpu.VMEM((BI,), jnp.int32),
          pltpu.VMEM((BI, 2 * D_MODEL), jnp.float32),
          pltpu.VMEM((BI, 2 * D_MODEL), jnp.float32),
          pltpu.VMEM((D_MODEL, BI), jnp.float32),
          pltpu.VMEM((D_MODEL, BI), jnp.float32),
          pltpu.SemaphoreType.DMA,
          pltpu.SemaphoreType.DMA,
          pltpu.SemaphoreType.DMA,
          pltpu.SemaphoreType.DMA,
      ],
      compiler_params=pltpu.CompilerParams(needs_layout_passes=False),
  )(functools.partial(_gather_body, b1))

  out3 = gather(xT, table_c)
  return out3.transpose(2, 0, 1)


# speedup vs baseline: 1.0607x; 1.0607x over previous
import functools
import math

import jax
import jax.numpy as jnp
from jax import lax
from jax.experimental import pallas as pl
from jax.experimental.pallas import tpu as pltpu
from jax.experimental.pallas import tpu_sc as plsc

D_MODEL = 64
SCALE = math.sqrt(D_MODEL)

NUM_CORES = 2
NUM_SUBCORES = 16
NW = NUM_CORES * NUM_SUBCORES

BI = 128
DEPTH = 4


def _gather_body(n_seq, xT_hbm, c_hbm, out_hbm,
                 xtb, wb0, wb1, wb2, wb3, g0, g1, g2, g3,
                 buf0, buf1, sg0, sg1, sg2, sg3, so0, so1):
  t = lax.axis_index("s") * NUM_CORES + lax.axis_index("c")
  pltpu.sync_copy(xT_hbm.at[:, pl.ds(t * BI, BI)], xtb)

  wbs = [wb0, wb1, wb2, wb3]
  gs = [g0, g1, g2, g3]
  sgs = [sg0, sg1, sg2, sg3]
  bufs = [buf0, buf1]
  sos = [so0, so1]

  iota = lax.iota(jnp.int32, 16)
  zero16 = iota * 0
  rvecs128 = [(i0 * 16 + iota) * (2 * D_MODEL) for i0 in range(8)]

  def prep(j, wb):
    for l in range(8):
      s = pl.ds(l * 16, 16)
      wb[s] = xtb[j, s] >> 1

  def fire_gather(wb, g, sem):
    pltpu.async_copy(c_hbm.at[wb], g, sem)

  def drain_gather(wb, g, sem):
    pltpu.make_async_copy(c_hbm.at[wb], g, sem).wait()

  def assemble(j, rows, buf):
    for i0 in range(8):
      pv = (xtb[j, pl.ds(i0 * 16, 16)] & 1) * D_MODEL
      base = rvecs128[i0] + pv
      si = pl.ds(i0 * 16, 16)

      @plsc.parallel_loop(0, D_MODEL, unroll=8)
      def _(k):
        buf[k, si] = plsc.load_gather(rows, [zero16, base + k]) * SCALE

  def fire_out(j, buf, sem):
    for k8 in range(8):
      pltpu.async_copy(buf.at[pl.ds(k8 * 8, 8)],
                       out_hbm.at[j, pl.ds(k8 * 8, 8), pl.ds(t * BI, BI)],
                       sem)

  def drain_out(j, buf, sem):
    for k8 in range(8):
      pltpu.make_async_copy(buf.at[pl.ds(k8 * 8, 8)],
                            out_hbm.at[j, pl.ds(k8 * 8, 8),
                                       pl.ds(t * BI, BI)], sem).wait()

  for j in range(DEPTH - 1):
    prep(j, wbs[j])
    fire_gather(wbs[j], gs[j], sgs[j])

  def quad(q, carry):
    for p in range(DEPTH):
      j = DEPTH * q + p
      pf = (p + DEPTH - 1) % DEPTH

      @pl.when(j + DEPTH - 1 < n_seq)
      def _():
        prep(j + DEPTH - 1, wbs[pf])
        fire_gather(wbs[pf], gs[pf], sgs[pf])

      drain_gather(wbs[p], gs[p], sgs[p])

      if p >= 2:
        drain_out(j - 2, bufs[p % 2], sos[p % 2])
      else:
        @pl.when(q > 0)
        def _():
          drain_out(j - 2, bufs[p % 2], sos[p % 2])

      assemble(j, gs[p], bufs[p % 2])
      fire_out(j, bufs[p % 2], sos[p % 2])
    return carry

  lax.fori_loop(0, n_seq // DEPTH, quad, 0)
  drain_out(n_seq - 2, bufs[0], sos[0])
  drain_out(n_seq - 1, bufs[1], sos[1])


def kernel(x, table):
  b0, b1 = x.shape
  v = table.shape[0]
  assert b0 == NW * BI and b1 % DEPTH == 0 and v % 2 == 0
  xT = x.T.astype(jnp.int32)
  table_c = table.reshape(v // 2, 2 * D_MODEL)

  mesh = plsc.VectorSubcoreMesh(core_axis_name="c", subcore_axis_name="s")
  gather = functools.partial(
      pl.kernel,
      mesh=mesh,
      out_type=jax.ShapeDtypeStruct((b1, D_MODEL, b0), jnp.float32),
      scratch_types=[
          pltpu.VMEM((b1, BI), jnp.int32),
          pltpu.VMEM((BI,), jnp.int32),
          pltpu.VMEM((BI,), jnp.int32),
          pltpu.VMEM((BI,), jnp.int32),
          pltpu.VMEM((BI,), jnp.int32),
          pltpu.VMEM((BI, 2 * D_MODEL), jnp.float32),
          pltpu.VMEM((BI, 2 * D_MODEL), jnp.float32),
          pltpu.VMEM((BI, 2 * D_MODEL), jnp.float32),
          pltpu.VMEM((BI, 2 * D_MODEL), jnp.float32),
          pltpu.VMEM((D_MODEL, BI), jnp.float32),
          pltpu.VMEM((D_MODEL, BI), jnp.float32),
          pltpu.SemaphoreType.DMA,
          pltpu.SemaphoreType.DMA,
          pltpu.SemaphoreType.DMA,
          pltpu.SemaphoreType.DMA,
          pltpu.SemaphoreType.DMA,
          pltpu.SemaphoreType.DMA,
      ],
      compiler_params=pltpu.CompilerParams(needs_layout_passes=False,
                                           disable_bounds_checks=True),
  )(functools.partial(_gather_body, b1))

  out3 = gather(xT, table_c)
  return out3.transpose(2, 0, 1)
